# baseline (speedup 1.0000x reference)
import functools

import jax
import jax.numpy as jnp
from jax import lax
from jax.experimental import pallas as pl
from jax.experimental.pallas import tpu as pltpu
from jax.experimental.pallas import tpu_sc as plsc

VOCAB = 100000
EMB = 128
FT = 64
OUT = EMB + FT
B = 4096
L = 200

BT = B * L
NC, NS = 2, 16
NW = NC * NS
PW = BT // NW
C = 128
NCHUNK = PW // C


def _sc_embed(idx2d, comb):
    mesh = plsc.VectorSubcoreMesh(core_axis_name="c", subcore_axis_name="s")

    @functools.partial(
        pl.kernel,
        mesh=mesh,
        out_type=jax.ShapeDtypeStruct((BT, OUT), jnp.float32),
        scratch_types=[
            pltpu.VMEM((NCHUNK, C), jnp.int32),
            pltpu.VMEM((C, OUT), jnp.float32),
            pltpu.VMEM((C, OUT), jnp.float32),
            pltpu.SemaphoreType.DMA,
            pltpu.SemaphoreType.DMA,
            pltpu.SemaphoreType.DMA,
            pltpu.SemaphoreType.DMA,
        ],
        compiler_params=pltpu.CompilerParams(use_tc_tiling_on_sc=False),
    )
    def k(idx_hbm, comb_hbm, out_hbm, idx_all, row0, row1, g0, g1, w0, w1):
        wid = lax.axis_index("s") * NC + lax.axis_index("c")
        elem_base = wid * PW
        rows = (row0, row1)
        gs = (g0, g1)
        ws = (w0, w1)

        pltpu.sync_copy(idx_hbm.at[pl.ds(wid * NCHUNK, NCHUNK)], idx_all)

        def fire_gather(i, b):
            pltpu.async_copy(comb_hbm.at[idx_all.at[i]], rows[b], gs[b])

        def wait_gather(i, b):
            pltpu.make_async_copy(comb_hbm.at[idx_all.at[i]],
                                  rows[b], gs[b]).wait()

        def fire_write(i, b):
            pltpu.async_copy(rows[b], out_hbm.at[pl.ds(elem_base + i * C, C)],
                             ws[b])

        def wait_write(i, b):
            pltpu.make_async_copy(rows[b],
                                  out_hbm.at[pl.ds(elem_base + i * C, C)],
                                  ws[b]).wait()

        fire_gather(0, 0)

        def pair(t, carry):
            for p in (0, 1):
                i = 2 * t + p
                b = p

                @pl.when(i > 0)
                def _():
                    wait_write(i - 1, 1 - b)

                @pl.when(i < NCHUNK - 1)
                def _():
                    fire_gather(i + 1, 1 - b)

                wait_gather(i, b)
                fire_write(i, b)
            return carry

        lax.fori_loop(0, NCHUNK // 2, pair, 0)
        wait_write(NCHUNK - 1, 1)

    return k(idx2d, comb)


def kernel(text, table, tune_table):
    idx2d = text.reshape(BT // C, C)
    comb = jnp.concatenate([table, tune_table], axis=1)
    out = _sc_embed(idx2d, comb)
    return out.reshape(B, L, OUT)


# speedup vs baseline: 1.5461x; 1.5461x over previous
import functools

import jax
import jax.numpy as jnp
from jax import lax
from jax.experimental import pallas as pl
from jax.experimental.pallas import tpu as pltpu
from jax.experimental.pallas import tpu_sc as plsc

VOCAB = 100000
EMB = 128
FT = 64
B = 4096
L = 200

BT = B * L
NC, NS = 2, 16
NW = NC * NS
PW = BT // NW
C = 128
NCHUNK = PW // C
VL = 16


def _sc_embed(idx2d, table, tune_pad):
    mesh = plsc.VectorSubcoreMesh(core_axis_name="c", subcore_axis_name="s")

    @functools.partial(
        pl.kernel,
        mesh=mesh,
        out_type=jax.ShapeDtypeStruct((BT, EMB + FT), jnp.float32),
        scratch_types=[
            pltpu.VMEM((NCHUNK, C), jnp.int32),
            pltpu.VMEM((C, EMB + FT), jnp.float32),
            pltpu.VMEM((C, EMB + FT), jnp.float32),
            pltpu.VMEM((C, EMB), jnp.float32),
            pltpu.VMEM((C, EMB), jnp.float32),
            pltpu.SemaphoreType.DMA,
            pltpu.SemaphoreType.DMA,
            pltpu.SemaphoreType.DMA,
            pltpu.SemaphoreType.DMA,
        ],
    )
    def k(idx_hbm, tab_hbm, tun_hbm, out_hbm, idx_all,
          row0, row1, tu0, tu1, g0, g1, w0, w1):
        wid = lax.axis_index("s") * NC + lax.axis_index("c")
        elem_base = wid * PW
        rows = (row0, row1)
        tu = (tu0, tu1)
        gs = (g0, g1)
        ws = (w0, w1)

        pltpu.sync_copy(idx_hbm.at[pl.ds(wid * NCHUNK, NCHUNK)], idx_all)

        def gather_args(i, b):
            return ((tab_hbm.at[idx_all.at[i]], rows[b].at[:, pl.ds(0, EMB)]),
                    (tun_hbm.at[idx_all.at[i]], tu[b]))

        def fire_gathers(i, b):
            for src, dst in gather_args(i, b):
                pltpu.async_copy(src, dst, gs[b])

        def wait_gathers(i, b):
            for src, dst in gather_args(i, b):
                pltpu.make_async_copy(src, dst, gs[b]).wait()

        def repack(b):
            def body(r, carry):
                for j in range(FT // VL):
                    rows[b][r, pl.ds(EMB + j * VL, VL)] = (
                        tu[b][r, pl.ds(j * VL, VL)])
                return carry
            lax.fori_loop(0, C, body, 0)

        def fire_write(i, b):
            pltpu.async_copy(rows[b], out_hbm.at[pl.ds(elem_base + i * C, C)],
                             ws[b])

        def wait_write(i, b):
            pltpu.make_async_copy(rows[b],
                                  out_hbm.at[pl.ds(elem_base + i * C, C)],
                                  ws[b]).wait()

        fire_gathers(0, 0)

        def pair(t, carry):
            for p in (0, 1):
                i = 2 * t + p
                b = p

                @pl.when(i > 0)
                def _():
                    wait_write(i - 1, 1 - b)

                @pl.when(i < NCHUNK - 1)
                def _():
                    fire_gathers(i + 1, 1 - b)

                wait_gathers(i, b)
                repack(b)
                fire_write(i, b)
            return carry

        lax.fori_loop(0, NCHUNK // 2, pair, 0)
        wait_write(NCHUNK - 1, 1)

    return k(idx2d, table, tune_pad)


def kernel(text, table, tune_table):
    idx2d = text.reshape(BT // C, C)
    tune_pad = jnp.pad(tune_table, ((0, 0), (0, EMB - FT)))
    out = _sc_embed(idx2d, table, tune_pad)
    return out.reshape(B, L, EMB + FT)
